# baseline (speedup 1.0000x reference)
import functools

import jax
import jax.numpy as jnp
from jax import lax
from jax.experimental import pallas as pl
from jax.experimental.pallas import tpu as pltpu
from jax.experimental.pallas import tpu_sc as plsc

B, T, D = 2, 2048, 1024
E, K, H = 8, 2, 2048
N = B * T
A = N * K
BLK = 256
NB = A // BLK + E
APAD = NB * BLK

_PREC = jax.lax.Precision.DEFAULT

NW = 32
NS = 16
GROWS = APAD // NW
GCH = 40
SCAT = A // NS
SCH = 128
WCH = APAD // NS
CTOK = N // NW
CCH = 16


def _router_body(x_ref, wg_ref, wa_ref, p_ref, be_ref, act_ref):
    x = x_ref[...]
    logits = lax.dot_general(
        x, wg_ref[...], (((1,), (1,)), ((), ())),
        preferred_element_type=jnp.float32, precision=_PREC)
    iota = lax.broadcasted_iota(jnp.int32, (N, E), 1)
    m1 = jnp.max(logits, axis=1, keepdims=True)
    i1 = jnp.min(jnp.where(logits == m1, iota, E), axis=1, keepdims=True)
    l2 = jnp.where(iota == i1, -jnp.inf, logits)
    m2 = jnp.max(l2, axis=1, keepdims=True)
    i2 = jnp.min(jnp.where(l2 == m2, iota, E), axis=1, keepdims=True)
    bexp = jnp.exp(m2 - m1)
    w1 = 1.0 / (1.0 + bexp)
    w2 = bexp * w1
    wa_ref[...] = jnp.concatenate([w1, w2], axis=0)

    ea = jnp.concatenate([i1, i2], axis=0)
    iota_a = lax.broadcasted_iota(jnp.int32, (A, E), 1)
    onehot = (iota_a == ea).astype(jnp.float32)
    SZ = 1024
    NSEG = A // SZ
    li = lax.broadcasted_iota(jnp.int32, (SZ, SZ), 0)
    lj = lax.broadcasted_iota(jnp.int32, (SZ, SZ), 1)
    lmask = (li >= lj).astype(jnp.float32)
    within = []
    lasts = []
    for s0 in range(NSEG):
        oh = onehot[s0 * SZ:(s0 + 1) * SZ]
        w = lax.dot_general(
            lmask, oh, (((1,), (0,)), ((), ())),
            preferred_element_type=jnp.float32)
        within.append(w)
        lasts.append(w[SZ - 1:SZ, :])
    seg_tot = jnp.concatenate(lasts, axis=0)
    s = 1
    segc = seg_tot
    while s < NSEG:
        segc = segc + jnp.concatenate(
            [jnp.zeros((s, E), jnp.float32), segc[:-s]], axis=0)
        s *= 2
    seg_off = segc - seg_tot
    c = jnp.concatenate(
        [within[s0] + seg_off[s0:s0 + 1, :] for s0 in range(NSEG)], axis=0)
    c = c.astype(jnp.int32)
    onehot = onehot.astype(jnp.int32)
    counts = c[A - 1:A, :]
    nblk = (counts + (BLK - 1)) // BLK
    cb = nblk
    s = 1
    while s < E:
        cb = cb + jnp.concatenate(
            [jnp.zeros((1, s), jnp.int32), cb[:, :-s]], axis=1)
        s *= 2
    blk_start = cb - nblk
    total_blocks = cb[:, E - 1:E]
    seg_start = blk_start * BLK

    rank = jnp.sum(c * onehot, axis=1, keepdims=True) - 1
    seg = jnp.sum(onehot * seg_start, axis=1, keepdims=True)
    p_ref[...] = seg + rank

    rowb = lax.broadcasted_iota(jnp.int32, (NB, E), 0)
    be_raw = jnp.sum((rowb >= cb).astype(jnp.int32), axis=1, keepdims=True)
    be_ref[...] = jnp.minimum(be_raw, E - 1)
    rowb1 = lax.broadcasted_iota(jnp.int32, (NB, 1), 0)
    act_ref[...] = (rowb1 < total_blocks).astype(jnp.int32)


def _route_tc(x2, Wg):
    return pl.pallas_call(
        _router_body,
        out_shape=[
            jax.ShapeDtypeStruct((A, 1), jnp.float32),
            jax.ShapeDtypeStruct((A, 1), jnp.int32),
            jax.ShapeDtypeStruct((NB, 1), jnp.int32),
            jax.ShapeDtypeStruct((NB, 1), jnp.int32),
        ],
        compiler_params=pltpu.CompilerParams(
            vmem_limit_bytes=100 * 1024 * 1024,
        ),
    )(x2, Wg)


def _permute_body(p_h, t_h, wa_h, zi_h, zf_h, x2_h,
                  xg_h, ws_h,
                  pv, tv, wav, wsv, idx0, idx1, rows0, rows1,
                  srid, sws, sem0, sem1):
    core = lax.axis_index("c")
    sub = lax.axis_index("s")

    @pl.when(sub == 0)
    def _init():
        pltpu.sync_copy(zi_h, srid)
        pltpu.sync_copy(zf_h, sws)

    plsc.subcore_barrier()

    abase = sub * SCAT
    pltpu.sync_copy(p_h.at[sub], pv)
    pltpu.sync_copy(t_h.at[pl.ds(abase, SCAT)], tv)
    pltpu.sync_copy(wa_h.at[pl.ds(abase, SCAT)], wav)
    for j in range(SCAT // SCH):
        sl = pl.ds(j * SCH, SCH)
        pltpu.sync_copy(tv.at[sl], srid.at[pv.at[j]], add=True)
        pltpu.sync_copy(wav.at[sl], sws.at[pv.at[j]], add=True)

    plsc.subcore_barrier()

    @pl.when(core == 0)
    def _ws_out():
        pltpu.sync_copy(sws.at[pl.ds(sub * WCH, WCH)], wsv)
        pltpu.sync_copy(wsv, ws_h.at[pl.ds(sub * WCH, WCH)])

    gwid = core * NS + sub
    base = gwid * GROWS
    nch = GROWS // GCH
    idx = (idx0, idx1)
    rows = (rows0, rows1)
    sems = (sem0, sem1)
    pltpu.sync_copy(srid.at[pl.ds(base, GCH)], idx0)
    cps = [pltpu.async_copy(x2_h.at[idx0], rows0, sem0)]
    for j in range(nch):
        cur = j % 2
        nxt = (j + 1) % 2
        if j + 1 < nch:
            pltpu.sync_copy(srid.at[pl.ds(base + (j + 1) * GCH, GCH)], idx[nxt])
            cps.append(pltpu.async_copy(x2_h.at[idx[nxt]], rows[nxt], sems[nxt]))
        cps[j].wait()
        pltpu.sync_copy(rows[cur], xg_h.at[pl.ds(base + j * GCH, GCH)])


def _permute_sc(p1d, t1d, wa1d, x2):
    mesh = plsc.VectorSubcoreMesh(core_axis_name="c", subcore_axis_name="s", num_cores=2, num_subcores=16)
    f = pl.kernel(
        _permute_body,
        out_type=[
            jax.ShapeDtypeStruct((APAD, D), jnp.float32),
            jax.ShapeDtypeStruct((APAD,), jnp.float32),
        ],
        mesh=mesh,
        scratch_types=[
            pltpu.VMEM((SCAT // SCH, SCH), jnp.int32),
            pltpu.VMEM((SCAT,), jnp.int32),
            pltpu.VMEM((SCAT,), jnp.float32),
            pltpu.VMEM((WCH,), jnp.float32),
            pltpu.VMEM((GCH,), jnp.int32),
            pltpu.VMEM((GCH,), jnp.int32),
            pltpu.VMEM((GCH, D), jnp.float32),
            pltpu.VMEM((GCH, D), jnp.float32),
            pltpu.VMEM_SHARED((APAD,), jnp.int32),
            pltpu.VMEM_SHARED((APAD,), jnp.float32),
            pltpu.SemaphoreType.DMA,
            pltpu.SemaphoreType.DMA,
        ],
        compiler_params=pltpu.CompilerParams(needs_layout_passes=False),
    )
    zi = jnp.zeros((APAD,), jnp.int32)
    zf = jnp.zeros((APAD,), jnp.float32)
    p3 = p1d.reshape(NS, SCAT // SCH, SCH)
    return f(p3, t1d, wa1d, zi, zf, x2)


def _gemm_body(be_ref, act_ref, xg_ref, wgate_ref, win_ref, wout_ref,
               ws_ref, out_ref):
    b = pl.program_id(0)

    @pl.when(act_ref[b] == 1)
    def _():
        x = xg_ref[...]
        g = lax.dot_general(
            x, wgate_ref[0], (((1,), (1,)), ((), ())),
            preferred_element_type=jnp.float32, precision=_PREC)
        u = lax.dot_general(
            x, win_ref[0], (((1,), (1,)), ((), ())),
            preferred_element_type=jnp.float32, precision=_PREC)
        h = g / (1.0 + jnp.exp(-g)) * u
        y = lax.dot_general(
            h, wout_ref[0], (((1,), (1,)), ((), ())),
            preferred_element_type=jnp.float32, precision=_PREC)
        out_ref[...] = y * ws_ref[...]


def _gemm_tc(be, act, xg, Wgate, Win, Wout, ws2):
    grid_spec = pltpu.PrefetchScalarGridSpec(
        num_scalar_prefetch=2,
        grid=(NB,),
        in_specs=[
            pl.BlockSpec((BLK, D), lambda b, be, act: (b, 0)),
            pl.BlockSpec((1, H, D), lambda b, be, act: (be[b], 0, 0)),
            pl.BlockSpec((1, H, D), lambda b, be, act: (be[b], 0, 0)),
            pl.BlockSpec((1, D, H), lambda b, be, act: (be[b], 0, 0)),
            pl.BlockSpec((BLK, 1), lambda b, be, act: (b, 0)),
        ],
        out_specs=pl.BlockSpec((BLK, D), lambda b, be, act: (b, 0)),
    )
    return pl.pallas_call(
        _gemm_body,
        grid_spec=grid_spec,
        out_shape=jax.ShapeDtypeStruct((APAD, D), jnp.float32),
        compiler_params=pltpu.CompilerParams(
            dimension_semantics=("arbitrary",),
            vmem_limit_bytes=100 * 1024 * 1024,
        ),
    )(be, act, xg, Wgate, Win, Wout, ws2)


def _combine_body(yg_h, p1_h, p2_h, out_h,
                  i1a, i1b, i2a, i2b, r1a, r1b, r2a, r2b, ov,
                  s1a, s1b, s2a, s2b):
    core = lax.axis_index("c")
    sub = lax.axis_index("s")
    gwid = core * NS + sub
    base = gwid * CTOK
    nch = CTOK // CCH
    i1 = (i1a, i1b)
    i2 = (i2a, i2b)
    r1 = (r1a, r1b)
    r2 = (r2a, r2b)
    s1 = (s1a, s1b)
    s2 = (s2a, s2b)

    def issue(j, buf):
        off = base + j * CCH
        pltpu.sync_copy(p1_h.at[pl.ds(off, CCH)], i1[buf])
        pltpu.sync_copy(p2_h.at[pl.ds(off, CCH)], i2[buf])
        return (pltpu.async_copy(yg_h.at[i1[buf]], r1[buf], s1[buf]),
                pltpu.async_copy(yg_h.at[i2[buf]], r2[buf], s2[buf]))

    cps = [issue(0, 0)]
    for j in range(nch):
        cur = j % 2
        if j + 1 < nch:
            cps.append(issue(j + 1, (j + 1) % 2))
        c1, c2 = cps[j]
        c1.wait()
        c2.wait()
        a, b = r1[cur], r2[cur]

        def body(i, _):
            def inner(cc, _):
                for u in range(4):
                    sl = pl.ds(cc * 64 + u * 16, 16)
                    ov[i, sl] = a[i, sl] + b[i, sl]
                return _
            return lax.fori_loop(0, D // 64, inner, _)

        lax.fori_loop(0, CCH, body, None)
        pltpu.sync_copy(ov, out_h.at[pl.ds(base + j * CCH, CCH)])


def _combine_sc(yg, pos1, pos2):
    mesh = plsc.VectorSubcoreMesh(core_axis_name="c", subcore_axis_name="s", num_cores=2, num_subcores=16)
    f = pl.kernel(
        _combine_body,
        out_type=jax.ShapeDtypeStruct((N, D), jnp.float32),
        mesh=mesh,
        scratch_types=[
            pltpu.VMEM((CCH,), jnp.int32),
            pltpu.VMEM((CCH,), jnp.int32),
            pltpu.VMEM((CCH,), jnp.int32),
            pltpu.VMEM((CCH,), jnp.int32),
            pltpu.VMEM((CCH, D), jnp.float32),
            pltpu.VMEM((CCH, D), jnp.float32),
            pltpu.VMEM((CCH, D), jnp.float32),
            pltpu.VMEM((CCH, D), jnp.float32),
            pltpu.VMEM((CCH, D), jnp.float32),
            pltpu.SemaphoreType.DMA,
            pltpu.SemaphoreType.DMA,
            pltpu.SemaphoreType.DMA,
            pltpu.SemaphoreType.DMA,
        ],
        compiler_params=pltpu.CompilerParams(needs_layout_passes=False),
    )
    return f(yg, pos1, pos2)


@jax.jit
def kernel(x, Wg, Wgate, Win, Wout):
    x2 = x.reshape(N, D)
    wa, p, be, act = _route_tc(x2, Wg)
    p1d = p.reshape(A)
    t1d = jnp.tile(jnp.arange(N, dtype=jnp.int32), K)
    xg, ws = _permute_sc(p1d, t1d, wa.reshape(A), x2)
    yg = _gemm_tc(be.reshape(NB), act.reshape(NB), xg,
                  Wgate, Win, Wout, ws.reshape(APAD, 1))
    out2 = _combine_sc(yg, p1d[:N], p1d[N:])
    return out2.reshape(B, T, D)


# speedup vs baseline: 1.0671x; 1.0671x over previous
import functools

import jax
import jax.numpy as jnp
from jax import lax
from jax.experimental import pallas as pl
from jax.experimental.pallas import tpu as pltpu
from jax.experimental.pallas import tpu_sc as plsc

B, T, D = 2, 2048, 1024
E, K, H = 8, 2, 2048
N = B * T
A = N * K
BLK = 256
NB = A // BLK + E
APAD = NB * BLK

_PREC = jax.lax.Precision.DEFAULT

NW = 32
NS = 16
GROWS = APAD // NW
GCH = 40
SCAT = A // NS
SCH = 128
WCH = APAD // NS
CTOK = N // NW
CCH = 16


def _router_body(x_ref, wg_ref, wa_ref, p_ref, be_ref, act_ref):
    x = x_ref[...]
    logits = lax.dot_general(
        x, wg_ref[...], (((1,), (1,)), ((), ())),
        preferred_element_type=jnp.float32, precision=_PREC)
    iota = lax.broadcasted_iota(jnp.int32, (N, E), 1)
    m1 = jnp.max(logits, axis=1, keepdims=True)
    i1 = jnp.min(jnp.where(logits == m1, iota, E), axis=1, keepdims=True)
    l2 = jnp.where(iota == i1, -jnp.inf, logits)
    m2 = jnp.max(l2, axis=1, keepdims=True)
    i2 = jnp.min(jnp.where(l2 == m2, iota, E), axis=1, keepdims=True)
    bexp = jnp.exp(m2 - m1)
    w1 = 1.0 / (1.0 + bexp)
    w2 = bexp * w1
    wa_ref[...] = jnp.concatenate([w1, w2], axis=0)

    ea = jnp.concatenate([i1, i2], axis=0)
    iota_a = lax.broadcasted_iota(jnp.int32, (A, E), 1)
    onehot = (iota_a == ea).astype(jnp.int32)
    c = onehot
    s = 1
    while s < A:
        c = c + jnp.concatenate(
            [jnp.zeros((s, E), jnp.int32), c[:-s]], axis=0)
        s *= 2
    counts = c[A - 1:A, :]
    nblk = (counts + (BLK - 1)) // BLK
    cb = nblk
    s = 1
    while s < E:
        cb = cb + jnp.concatenate(
            [jnp.zeros((1, s), jnp.int32), cb[:, :-s]], axis=1)
        s *= 2
    blk_start = cb - nblk
    total_blocks = cb[:, E - 1:E]
    seg_start = blk_start * BLK

    rank = jnp.sum(c * onehot, axis=1, keepdims=True) - 1
    seg = jnp.sum(onehot * seg_start, axis=1, keepdims=True)
    p_ref[...] = seg + rank

    rowb = lax.broadcasted_iota(jnp.int32, (NB, E), 0)
    be_raw = jnp.sum((rowb >= cb).astype(jnp.int32), axis=1, keepdims=True)
    be_ref[...] = jnp.minimum(be_raw, E - 1)
    rowb1 = lax.broadcasted_iota(jnp.int32, (NB, 1), 0)
    act_ref[...] = (rowb1 < total_blocks).astype(jnp.int32)


def _route_tc(x2, Wg):
    return pl.pallas_call(
        _router_body,
        out_shape=[
            jax.ShapeDtypeStruct((A, 1), jnp.float32),
            jax.ShapeDtypeStruct((A, 1), jnp.int32),
            jax.ShapeDtypeStruct((NB, 1), jnp.int32),
            jax.ShapeDtypeStruct((NB, 1), jnp.int32),
        ],
        compiler_params=pltpu.CompilerParams(
            vmem_limit_bytes=100 * 1024 * 1024,
        ),
    )(x2, Wg)


HALF = APAD // 2
HROWS = HALF // NW
NBH = NB // 2


def _scatter_body(p_h, t_h, wa_h, zi_h, zf_h,
                  rid_h, ws_h,
                  pv, tv, wav, wsv, riv, srid, sws):
    core = lax.axis_index("c")
    sub = lax.axis_index("s")

    @pl.when(sub == 0)
    def _init():
        pltpu.sync_copy(zi_h, srid)
        pltpu.sync_copy(zf_h, sws)

    plsc.subcore_barrier()

    abase = sub * SCAT
    pltpu.sync_copy(p_h.at[sub], pv)
    pltpu.sync_copy(t_h.at[pl.ds(abase, SCAT)], tv)
    pltpu.sync_copy(wa_h.at[pl.ds(abase, SCAT)], wav)
    for j in range(SCAT // SCH):
        sl = pl.ds(j * SCH, SCH)
        pltpu.sync_copy(tv.at[sl], srid.at[pv.at[j]], add=True)
        pltpu.sync_copy(wav.at[sl], sws.at[pv.at[j]], add=True)

    plsc.subcore_barrier()

    @pl.when(core == 0)
    def _ws_out():
        pltpu.sync_copy(sws.at[pl.ds(sub * WCH, WCH)], wsv)
        pltpu.sync_copy(wsv, ws_h.at[pl.ds(sub * WCH, WCH)])

    @pl.when(core == 1)
    def _rid_out():
        pltpu.sync_copy(srid.at[pl.ds(sub * WCH, WCH)], riv)
        pltpu.sync_copy(riv, rid_h.at[pl.ds(sub * WCH, WCH)])


def _scatter_sc(p1d, t1d, wa1d):
    mesh = plsc.VectorSubcoreMesh(core_axis_name="c", subcore_axis_name="s", num_cores=2, num_subcores=16)
    f = pl.kernel(
        _scatter_body,
        out_type=[
            jax.ShapeDtypeStruct((APAD,), jnp.int32),
            jax.ShapeDtypeStruct((APAD,), jnp.float32),
        ],
        mesh=mesh,
        scratch_types=[
            pltpu.VMEM((SCAT // SCH, SCH), jnp.int32),
            pltpu.VMEM((SCAT,), jnp.int32),
            pltpu.VMEM((SCAT,), jnp.float32),
            pltpu.VMEM((WCH,), jnp.float32),
            pltpu.VMEM((WCH,), jnp.int32),
            pltpu.VMEM_SHARED((APAD,), jnp.int32),
            pltpu.VMEM_SHARED((APAD,), jnp.float32),
        ],
        compiler_params=pltpu.CompilerParams(needs_layout_passes=False),
    )
    zi = jnp.zeros((APAD,), jnp.int32)
    zf = jnp.zeros((APAD,), jnp.float32)
    p3 = p1d.reshape(NS, SCAT // SCH, SCH)
    return f(p3, t1d, wa1d, zi, zf)


def _gather_body(start, rid_h, x2_h, xg_h,
                 idx0, idx1, rows0, rows1, sem0, sem1):
    core = lax.axis_index("c")
    sub = lax.axis_index("s")
    gwid = core * NS + sub
    base = gwid * HROWS
    nch = HROWS // GCH
    idx = (idx0, idx1)
    rows = (rows0, rows1)
    sems = (sem0, sem1)
    pltpu.sync_copy(rid_h.at[pl.ds(start + base, GCH)], idx0)
    cps = [pltpu.async_copy(x2_h.at[idx0], rows0, sem0)]
    for j in range(nch):
        cur = j % 2
        nxt = (j + 1) % 2
        if j + 1 < nch:
            pltpu.sync_copy(
                rid_h.at[pl.ds(start + base + (j + 1) * GCH, GCH)], idx[nxt])
            cps.append(pltpu.async_copy(x2_h.at[idx[nxt]], rows[nxt], sems[nxt]))
        cps[j].wait()
        pltpu.sync_copy(rows[cur], xg_h.at[pl.ds(base + j * GCH, GCH)])


def _gather_sc(rid, x2, start):
    mesh = plsc.VectorSubcoreMesh(core_axis_name="c", subcore_axis_name="s", num_cores=2, num_subcores=16)
    f = pl.kernel(
        functools.partial(_gather_body, start),
        out_type=jax.ShapeDtypeStruct((HALF, D), jnp.float32),
        mesh=mesh,
        scratch_types=[
            pltpu.VMEM((GCH,), jnp.int32),
            pltpu.VMEM((GCH,), jnp.int32),
            pltpu.VMEM((GCH, D), jnp.float32),
            pltpu.VMEM((GCH, D), jnp.float32),
            pltpu.SemaphoreType.DMA,
            pltpu.SemaphoreType.DMA,
        ],
        compiler_params=pltpu.CompilerParams(needs_layout_passes=False),
    )
    return f(rid, x2)


def _gemm_body_first(be_ref, act_ref, xg_ref, wgate_ref, win_ref, wout_ref,
                     ws_ref, out_ref):
    _gemm_common(0, be_ref, act_ref, xg_ref, wgate_ref, win_ref, wout_ref,
                 ws_ref, out_ref)


def _gemm_body_second(be_ref, act_ref, prev_ref, xg_ref, wgate_ref, win_ref,
                      wout_ref, ws_ref, out_ref):
    _gemm_common(NBH, be_ref, act_ref, xg_ref, wgate_ref, win_ref, wout_ref,
                 ws_ref, out_ref)


def _gemm_common(bo, be_ref, act_ref, xg_ref, wgate_ref, win_ref, wout_ref,
                 ws_ref, out_ref):
    b = pl.program_id(0)

    @pl.when(act_ref[b + bo] == 1)
    def _():
        x = xg_ref[...]
        g = lax.dot_general(
            x, wgate_ref[0], (((1,), (1,)), ((), ())),
            preferred_element_type=jnp.float32, precision=_PREC)
        u = lax.dot_general(
            x, win_ref[0], (((1,), (1,)), ((), ())),
            preferred_element_type=jnp.float32, precision=_PREC)
        h = g / (1.0 + jnp.exp(-g)) * u
        y = lax.dot_general(
            h, wout_ref[0], (((1,), (1,)), ((), ())),
            preferred_element_type=jnp.float32, precision=_PREC)
        out_ref[...] = y * ws_ref[...]


def _gemm_tc(be, act, xg, Wgate, Win, Wout, ws2, yg_prev):
    bo = 0 if yg_prev is None else NBH
    wmap = lambda b, be, act: (be[b + bo], 0, 0)
    in_specs = [
        pl.BlockSpec((BLK, D), lambda b, be, act: (b, 0)),
        pl.BlockSpec((1, H, D), wmap),
        pl.BlockSpec((1, H, D), wmap),
        pl.BlockSpec((1, D, H), wmap),
        pl.BlockSpec((BLK, 1), lambda b, be, act: (b + bo, 0)),
    ]
    args = [be, act, xg, Wgate, Win, Wout, ws2]
    body = _gemm_body_first
    aliases = {}
    if yg_prev is not None:
        in_specs.insert(0, pl.BlockSpec((BLK, D), lambda b, be, act: (0, 0)))
        args.insert(2, yg_prev)
        body = _gemm_body_second
        aliases = {2: 0}
    grid_spec = pltpu.PrefetchScalarGridSpec(
        num_scalar_prefetch=2,
        grid=(NBH,),
        in_specs=in_specs,
        out_specs=pl.BlockSpec((BLK, D), lambda b, be, act: (b + bo, 0)),
    )
    return pl.pallas_call(
        body,
        grid_spec=grid_spec,
        out_shape=jax.ShapeDtypeStruct((APAD, D), jnp.float32),
        input_output_aliases=aliases,
        compiler_params=pltpu.CompilerParams(
            dimension_semantics=("arbitrary",),
            vmem_limit_bytes=100 * 1024 * 1024,
        ),
    )(*args)


def _combine_body(yg_h, p1_h, p2_h, out_h,
                  i1a, i1b, i2a, i2b, r1a, r1b, r2a, r2b, ov,
                  s1a, s1b, s2a, s2b):
    core = lax.axis_index("c")
    sub = lax.axis_index("s")
    gwid = core * NS + sub
    base = gwid * CTOK
    nch = CTOK // CCH
    i1 = (i1a, i1b)
    i2 = (i2a, i2b)
    r1 = (r1a, r1b)
    r2 = (r2a, r2b)
    s1 = (s1a, s1b)
    s2 = (s2a, s2b)

    def issue(j, buf):
        off = base + j * CCH
        pltpu.sync_copy(p1_h.at[pl.ds(off, CCH)], i1[buf])
        pltpu.sync_copy(p2_h.at[pl.ds(off, CCH)], i2[buf])
        return (pltpu.async_copy(yg_h.at[i1[buf]], r1[buf], s1[buf]),
                pltpu.async_copy(yg_h.at[i2[buf]], r2[buf], s2[buf]))

    cps = [issue(0, 0)]
    for j in range(nch):
        cur = j % 2
        if j + 1 < nch:
            cps.append(issue(j + 1, (j + 1) % 2))
        c1, c2 = cps[j]
        c1.wait()
        c2.wait()
        a, b = r1[cur], r2[cur]

        def body(i, _):
            def inner(cc, _):
                for u in range(4):
                    sl = pl.ds(cc * 64 + u * 16, 16)
                    ov[i, sl] = a[i, sl] + b[i, sl]
                return _
            return lax.fori_loop(0, D // 64, inner, _)

        lax.fori_loop(0, CCH, body, None)
        pltpu.sync_copy(ov, out_h.at[pl.ds(base + j * CCH, CCH)])


def _combine_sc(yg, pos1, pos2):
    mesh = plsc.VectorSubcoreMesh(core_axis_name="c", subcore_axis_name="s", num_cores=2, num_subcores=16)
    f = pl.kernel(
        _combine_body,
        out_type=jax.ShapeDtypeStruct((N, D), jnp.float32),
        mesh=mesh,
        scratch_types=[
            pltpu.VMEM((CCH,), jnp.int32),
            pltpu.VMEM((CCH,), jnp.int32),
            pltpu.VMEM((CCH,), jnp.int32),
            pltpu.VMEM((CCH,), jnp.int32),
            pltpu.VMEM((CCH, D), jnp.float32),
            pltpu.VMEM((CCH, D), jnp.float32),
            pltpu.VMEM((CCH, D), jnp.float32),
            pltpu.VMEM((CCH, D), jnp.float32),
            pltpu.VMEM((CCH, D), jnp.float32),
            pltpu.SemaphoreType.DMA,
            pltpu.SemaphoreType.DMA,
            pltpu.SemaphoreType.DMA,
            pltpu.SemaphoreType.DMA,
        ],
        compiler_params=pltpu.CompilerParams(needs_layout_passes=False),
    )
    return f(yg, pos1, pos2)


@jax.jit
def kernel(x, Wg, Wgate, Win, Wout):
    x2 = x.reshape(N, D)
    wa, p, be, act = _route_tc(x2, Wg)
    p1d = p.reshape(A)
    t1d = jnp.tile(jnp.arange(N, dtype=jnp.int32), K)
    rid, ws = _scatter_sc(p1d, t1d, wa.reshape(A))
    xg1 = _gather_sc(rid, x2, 0)
    xg2 = _gather_sc(rid, x2, HALF)
    be1 = be.reshape(NB)
    act1 = act.reshape(NB)
    ws2 = ws.reshape(APAD, 1)
    yg1 = _gemm_tc(be1, act1, xg1, Wgate, Win, Wout, ws2, None)
    yg = _gemm_tc(be1, act1, xg2, Wgate, Win, Wout, ws2, yg1)
    out2 = _combine_sc(yg, p1d[:N], p1d[N:])
    return out2.reshape(B, T, D)
